# Initial kernel scaffold; baseline (speedup 1.0000x reference)
import functools

import jax
import jax.numpy as jnp
from jax import lax
from jax.experimental import pallas as pl
from jax.experimental.pallas import tpu as pltpu
from jax.experimental.pallas import tpu_sc as plsc

N_NODES = 10000
D = 128
E = 160000
ROWS_PAD = 10240
DEG_W = 8
CHUNK = 128
NC = 2
NS = 16
N_CHUNKS = E // CHUNK
CH_PER_CORE = N_CHUNKS // NC
ROWS_PER_TILE = ROWS_PAD // NS
K_FLUSH = ROWS_PER_TILE // CHUNK
NSLOT = 3 * NC


def _sc_body(fu, fi, c_src, c_dst, cb_src, cb_dst, fo_src, fo_dst,
             z128, z8, o8, sums_h, degs_h,
             accum, degacc, zbuf, zdeg, ones_v, sidx, didx, rows, sem):
    cid = lax.axis_index("c")
    sid = lax.axis_index("s")
    pltpu.sync_copy(z128, zbuf)
    pltpu.sync_copy(z8, zdeg)
    pltpu.sync_copy(o8, ones_v)
    etypes = ((fu, c_src, c_dst), (fi, cb_src, cb_dst), (fu, fo_src, fo_dst))
    for e, (tab, src, dst) in enumerate(etypes):
        slot = e * NC + cid
        for k in range(K_FLUSH):
            r0 = sid * ROWS_PER_TILE + k * CHUNK
            pltpu.sync_copy(zbuf, accum.at[pl.ds(r0, CHUNK)])
            pltpu.sync_copy(zdeg, degacc.at[pl.ds(r0, CHUNK)])
        plsc.subcore_barrier()

        n_iter = (CH_PER_CORE - sid + NS - 1) // NS

        def chunk_step(i, carry):
            c = cid * CH_PER_CORE + sid + i * NS
            base = c * CHUNK
            pltpu.sync_copy(src.at[pl.ds(base, CHUNK)], sidx)
            pltpu.sync_copy(dst.at[pl.ds(base, CHUNK)], didx)
            pltpu.async_copy(tab.at[sidx], rows, sem).wait()
            pltpu.sync_copy(rows, accum.at[didx], add=True)
            pltpu.sync_copy(ones_v, degacc.at[didx], add=True)
            return carry

        lax.fori_loop(0, n_iter, chunk_step, 0)
        plsc.subcore_barrier()

        for k in range(K_FLUSH):
            r0 = sid * ROWS_PER_TILE + k * CHUNK
            out_r0 = slot * ROWS_PAD + r0
            pltpu.sync_copy(accum.at[pl.ds(r0, CHUNK)],
                            sums_h.at[pl.ds(out_r0, CHUNK)])
            pltpu.sync_copy(degacc.at[pl.ds(r0, CHUNK)],
                            degs_h.at[pl.ds(out_r0, CHUNK)])
        plsc.subcore_barrier()


_phase1 = pl.kernel(
    _sc_body,
    out_type=(
        jax.ShapeDtypeStruct((NSLOT * ROWS_PAD, D), jnp.float32),
        jax.ShapeDtypeStruct((NSLOT * ROWS_PAD, DEG_W), jnp.float32),
    ),
    mesh=plsc.VectorSubcoreMesh(core_axis_name="c", subcore_axis_name="s"),
    scratch_types=[
        pltpu.VMEM_SHARED((ROWS_PAD, D), jnp.float32),
        pltpu.VMEM_SHARED((ROWS_PAD, DEG_W), jnp.float32),
        pltpu.VMEM((CHUNK, D), jnp.float32),
        pltpu.VMEM((CHUNK, DEG_W), jnp.float32),
        pltpu.VMEM((CHUNK, DEG_W), jnp.float32),
        pltpu.VMEM((CHUNK,), jnp.int32),
        pltpu.VMEM((CHUNK,), jnp.int32),
        pltpu.VMEM((CHUNK, D), jnp.float32),
        pltpu.SemaphoreType.DMA,
    ],
)


def _tc_body(s_ref, d_ref, w_ref, b_ref, hu_ref, hi_ref):
    def term(e):
        s = s_ref[2 * e, 0] + s_ref[2 * e + 1, 0]
        d = d_ref[2 * e] + d_ref[2 * e + 1]
        m = s * (1.0 / jnp.maximum(d, 1.0))
        out = jnp.dot(m, w_ref[e], preferred_element_type=jnp.float32)
        return out + (d > 0.0).astype(jnp.float32) * b_ref[e][None, :]

    hi_ref[...] = term(0)
    hu_ref[...] = term(1) + term(2)


_phase2 = pl.pallas_call(
    _tc_body,
    grid=(ROWS_PAD // 128,),
    in_specs=[
        pl.BlockSpec((NSLOT, 1, 128, D), lambda b: (0, b, 0, 0)),
        pl.BlockSpec((NSLOT, 128, 1), lambda b: (0, b, 0)),
        pl.BlockSpec((3, D, D), lambda b: (0, 0, 0)),
        pl.BlockSpec((3, D), lambda b: (0, 0)),
    ],
    out_specs=[
        pl.BlockSpec((128, D), lambda b: (b, 0)),
        pl.BlockSpec((128, D), lambda b: (b, 0)),
    ],
    out_shape=[
        jax.ShapeDtypeStruct((ROWS_PAD, D), jnp.float32),
        jax.ShapeDtypeStruct((ROWS_PAD, D), jnp.float32),
    ],
)


def kernel(feat_user, feat_item, clicks_src, clicks_dst, clicked_by_src,
           clicked_by_dst, follows_src, follows_dst, W_clicks, b_clicks,
           W_clicked_by, b_clicked_by, W_follows, b_follows):
    i32 = lambda x: x.astype(jnp.int32)
    z128 = jnp.zeros((CHUNK, D), jnp.float32)
    z8 = jnp.zeros((CHUNK, DEG_W), jnp.float32)
    o8 = jnp.ones((CHUNK, DEG_W), jnp.float32)
    sums, degs = _phase1(
        feat_user, feat_item, i32(clicks_src), i32(clicks_dst),
        i32(clicked_by_src), i32(clicked_by_dst),
        i32(follows_src), i32(follows_dst), z128, z8, o8)
    sums4 = sums.reshape(NSLOT, ROWS_PAD // 128, 128, D)
    degs3 = degs[:, :1].reshape(NSLOT, ROWS_PAD, 1)
    wstk = jnp.stack([W_clicks, W_clicked_by, W_follows])
    bstk = jnp.stack([b_clicks, b_clicked_by, b_follows])
    h_user, h_item = _phase2(sums4, degs3, wstk, bstk)
    return (h_user[:N_NODES], h_item[:N_NODES])


# speedup vs baseline: 3.5182x; 3.5182x over previous
import jax
import jax.numpy as jnp
from jax import lax
from jax.experimental import pallas as pl
from jax.experimental.pallas import tpu as pltpu
from jax.experimental.pallas import tpu_sc as plsc

N_NODES = 10000
D = 128
DH = D // 2
E = 160000
ROWS_PAD = 10240
DEG_W = 8
CHUNK = 128
NC = 2
NS = 16
N_CHUNKS = E // CHUNK
ROWS_PER_TILE = ROWS_PAD // NS
K_FLUSH = ROWS_PER_TILE // CHUNK
NSLOT = 3 * NC


def _sc_body(fu, fi, c_src, c_dst, cb_src, cb_dst, fo_src, fo_dst,
             zh, z8, o8, sums_h, degs_h,
             accum, degacc, zbuf, zdeg, ones_v, sidx, didx, rows, sem):
    cid = lax.axis_index("c")
    sid = lax.axis_index("s")
    pltpu.sync_copy(zh, zbuf)
    pltpu.sync_copy(z8, zdeg)
    pltpu.sync_copy(o8, ones_v)
    half_off = cid * N_NODES
    etypes = ((fu, c_src, c_dst), (fi, cb_src, cb_dst), (fu, fo_src, fo_dst))
    for e, (tab, src, dst) in enumerate(etypes):
        slot = e * NC + cid
        for k in range(K_FLUSH):
            r0 = sid * ROWS_PER_TILE + k * CHUNK
            pltpu.sync_copy(zbuf, accum.at[pl.ds(r0, CHUNK)])
            pltpu.sync_copy(zdeg, degacc.at[pl.ds(r0, CHUNK)])
        plsc.subcore_barrier()

        n_iter = (N_CHUNKS - sid + NS - 1) // NS

        def chunk_step(i, carry):
            base = (sid + i * NS) * CHUNK
            pltpu.sync_copy(src.at[pl.ds(base, CHUNK)], sidx)
            pltpu.sync_copy(dst.at[pl.ds(base, CHUNK)], didx)
            for j in range(CHUNK // 16):
                v = sidx[pl.ds(j * 16, 16)]
                sidx[pl.ds(j * 16, 16)] = v + half_off
            pltpu.async_copy(tab.at[sidx], rows, sem).wait()
            pltpu.sync_copy(rows, accum.at[didx], add=True)
            pltpu.sync_copy(ones_v, degacc.at[didx], add=True)
            return carry

        lax.fori_loop(0, n_iter, chunk_step, 0)
        plsc.subcore_barrier()

        for k in range(K_FLUSH):
            r0 = sid * ROWS_PER_TILE + k * CHUNK
            out_r0 = slot * ROWS_PAD + r0
            pltpu.sync_copy(accum.at[pl.ds(r0, CHUNK)],
                            sums_h.at[pl.ds(out_r0, CHUNK)])
            pltpu.sync_copy(degacc.at[pl.ds(r0, CHUNK)],
                            degs_h.at[pl.ds(out_r0, CHUNK)])
        plsc.subcore_barrier()


_phase1 = pl.kernel(
    _sc_body,
    out_type=(
        jax.ShapeDtypeStruct((NSLOT * ROWS_PAD, DH), jnp.float32),
        jax.ShapeDtypeStruct((NSLOT * ROWS_PAD, DEG_W), jnp.float32),
    ),
    mesh=plsc.VectorSubcoreMesh(core_axis_name="c", subcore_axis_name="s"),
    compiler_params=pltpu.CompilerParams(use_tc_tiling_on_sc=False),
    scratch_types=[
        pltpu.VMEM_SHARED((ROWS_PAD, DH), jnp.float32),
        pltpu.VMEM_SHARED((ROWS_PAD, DEG_W), jnp.float32),
        pltpu.VMEM((CHUNK, DH), jnp.float32),
        pltpu.VMEM((CHUNK, DEG_W), jnp.float32),
        pltpu.VMEM((CHUNK, DEG_W), jnp.float32),
        pltpu.VMEM((CHUNK,), jnp.int32),
        pltpu.VMEM((CHUNK,), jnp.int32),
        pltpu.VMEM((CHUNK, DH), jnp.float32),
        pltpu.SemaphoreType.DMA,
    ],
)


def _tc_body(s_ref, d_ref, w_ref, b_ref, hu_ref, hi_ref):
    def term(e):
        d = d_ref[2 * e]
        inv = 1.0 / jnp.maximum(d, 1.0)
        m_lo = s_ref[2 * e, 0] * inv
        m_hi = s_ref[2 * e + 1, 0] * inv
        out = jnp.dot(m_lo, w_ref[e, :DH, :],
                      preferred_element_type=jnp.float32)
        out += jnp.dot(m_hi, w_ref[e, DH:, :],
                       preferred_element_type=jnp.float32)
        return out + (d > 0.0).astype(jnp.float32) * b_ref[e][None, :]

    hi_ref[...] = term(0)
    hu_ref[...] = term(1) + term(2)


_phase2 = pl.pallas_call(
    _tc_body,
    grid=(ROWS_PAD // 128,),
    in_specs=[
        pl.BlockSpec((NSLOT, 1, 128, DH), lambda b: (0, b, 0, 0)),
        pl.BlockSpec((NSLOT, 128, 1), lambda b: (0, b, 0)),
        pl.BlockSpec((3, D, D), lambda b: (0, 0, 0)),
        pl.BlockSpec((3, D), lambda b: (0, 0)),
    ],
    out_specs=[
        pl.BlockSpec((128, D), lambda b: (b, 0)),
        pl.BlockSpec((128, D), lambda b: (b, 0)),
    ],
    out_shape=[
        jax.ShapeDtypeStruct((ROWS_PAD, D), jnp.float32),
        jax.ShapeDtypeStruct((ROWS_PAD, D), jnp.float32),
    ],
)


def _split_cols(feat):
    return feat.reshape(-1, NC, DH).transpose(1, 0, 2).reshape(-1, DH)


def kernel(feat_user, feat_item, clicks_src, clicks_dst, clicked_by_src,
           clicked_by_dst, follows_src, follows_dst, W_clicks, b_clicks,
           W_clicked_by, b_clicked_by, W_follows, b_follows):
    i32 = lambda x: x.astype(jnp.int32)
    zh = jnp.zeros((CHUNK, DH), jnp.float32)
    z8 = jnp.zeros((CHUNK, DEG_W), jnp.float32)
    o8 = jnp.ones((CHUNK, DEG_W), jnp.float32)
    sums, degs = _phase1(
        _split_cols(feat_user), _split_cols(feat_item),
        i32(clicks_src), i32(clicks_dst),
        i32(clicked_by_src), i32(clicked_by_dst),
        i32(follows_src), i32(follows_dst), zh, z8, o8)
    sums4 = sums.reshape(NSLOT, ROWS_PAD // 128, 128, DH)
    degs3 = degs[:, :1].reshape(NSLOT, ROWS_PAD, 1)
    wstk = jnp.stack([W_clicks, W_clicked_by, W_follows])
    bstk = jnp.stack([b_clicks, b_clicked_by, b_follows])
    h_user, h_item = _phase2(sums4, degs3, wstk, bstk)
    return (h_user[:N_NODES], h_item[:N_NODES])


# speedup vs baseline: 4.0858x; 1.1613x over previous
import jax
import jax.numpy as jnp
from jax import lax
from jax.experimental import pallas as pl
from jax.experimental.pallas import tpu as pltpu
from jax.experimental.pallas import tpu_sc as plsc

N_NODES = 10000
D = 128
DH = D // 2
E = 160000
ROWS_PAD = 10240
DEG_W = 8
CHUNK = 128
NC = 2
NS = 16
N_CHUNKS = E // CHUNK
MAXCH = 79
NBUF_ROWS = 80
K = 4
NSUP = NBUF_ROWS // K
ROWS_PER_TILE = ROWS_PAD // NS
K_FLUSH = ROWS_PER_TILE // CHUNK
NSLOT = 3 * NC
VECS = CHUNK // 16


def _sc_body(fu, fi, c_src, c_dst, cb_src, cb_dst, fo_src, fo_dst,
             zh, z8, o4, sums_h, degs_h,
             accum, degacc, zbuf, zdeg, ones4, sidx, didx, rows0, rows1,
             gsem0, gsem1, ssem0, ssem1):
    cid = lax.axis_index("c")
    sid = lax.axis_index("s")
    pltpu.sync_copy(zh, zbuf)
    pltpu.sync_copy(z8, zdeg)
    pltpu.sync_copy(o4, ones4)
    half_off = cid * N_NODES
    c0 = sid * N_CHUNKS // NS
    n_ch = (sid + 1) * N_CHUNKS // NS - c0
    pad_dst = N_NODES + sid * 8

    etypes = ((fu, c_src, c_dst), (fi, cb_src, cb_dst), (fu, fo_src, fo_dst))
    for e, (tab, src, dst) in enumerate(etypes):
        slot = e * NC + cid
        for k in range(K_FLUSH):
            r0 = sid * ROWS_PER_TILE + k * CHUNK
            pltpu.sync_copy(zbuf, accum.at[pl.ds(r0, CHUNK)])
            pltpu.sync_copy(zdeg, degacc.at[pl.ds(r0, CHUNK)])

        pltpu.sync_copy(src.at[pl.ds(c0, MAXCH)], sidx.at[pl.ds(0, MAXCH)])
        pltpu.sync_copy(dst.at[pl.ds(c0, MAXCH)], didx.at[pl.ds(0, MAXCH)])
        for r in (MAXCH - 1, MAXCH):
            @pl.when(r >= n_ch)
            def _():
                for j in range(VECS):
                    sidx[r, pl.ds(j * 16, 16)] = jnp.zeros((16,), jnp.int32)
                    didx[r, pl.ds(j * 16, 16)] = jnp.full((16,), pad_dst,
                                                          jnp.int32)

        def adj(r, carry):
            for j in range(VECS):
                v = sidx[r, pl.ds(j * 16, 16)]
                sidx[r, pl.ds(j * 16, 16)] = v + half_off
            return carry

        lax.fori_loop(0, NBUF_ROWS, adj, 0)
        plsc.subcore_barrier()

        def gather(j, buf, sem):
            pltpu.async_copy(tab.at[sidx.at[j]], buf, sem)

        def gwait(j, buf, sem):
            pltpu.make_async_copy(tab.at[sidx.at[j]], buf, sem).wait()

        def scat(j, buf, sem):
            pltpu.async_copy(buf, accum.at[didx.at[j]], sem, add=True)
            pltpu.async_copy(ones4, degacc.at[didx.at[j]], sem, add=True)

        def swait(j, buf, sem):
            pltpu.make_async_copy(buf, accum.at[didx.at[j]], sem).wait()
            pltpu.make_async_copy(ones4, degacc.at[didx.at[j]], sem).wait()

        gather(0, rows0, gsem0)

        def pair(p, carry):
            j0 = 2 * p
            j1 = j0 + 1
            gwait(j0, rows0, gsem0)
            scat(j0, rows0, ssem0)

            @pl.when(p > 0)
            def _():
                swait(j1 - 2, rows1, ssem1)

            gather(j1, rows1, gsem1)
            gwait(j1, rows1, gsem1)
            scat(j1, rows1, ssem1)

            @pl.when(p < NBUF_ROWS // 2 - 1)
            def _():
                swait(j0, rows0, ssem0)
                gather(j0 + 2, rows0, gsem0)

            return carry

        lax.fori_loop(0, NBUF_ROWS // 2, pair, 0)
        swait(NBUF_ROWS - 2, rows0, ssem0)
        swait(NBUF_ROWS - 1, rows1, ssem1)
        plsc.subcore_barrier()

        r0 = sid * ROWS_PER_TILE
        out_r0 = slot * ROWS_PAD + r0
        pltpu.sync_copy(accum.at[pl.ds(r0, ROWS_PER_TILE)],
                        sums_h.at[pl.ds(out_r0, ROWS_PER_TILE)])
        pltpu.sync_copy(degacc.at[pl.ds(r0, ROWS_PER_TILE)],
                        degs_h.at[pl.ds(out_r0, ROWS_PER_TILE)])
        plsc.subcore_barrier()


_phase1 = pl.kernel(
    _sc_body,
    out_type=(
        jax.ShapeDtypeStruct((NSLOT * ROWS_PAD, DH), jnp.float32),
        jax.ShapeDtypeStruct((NSLOT * ROWS_PAD, DEG_W), jnp.float32),
    ),
    mesh=plsc.VectorSubcoreMesh(core_axis_name="c", subcore_axis_name="s"),
    compiler_params=pltpu.CompilerParams(use_tc_tiling_on_sc=False),
    scratch_types=[
        pltpu.VMEM_SHARED((ROWS_PAD, DH), jnp.float32),
        pltpu.VMEM_SHARED((ROWS_PAD, DEG_W), jnp.float32),
        pltpu.VMEM((CHUNK, DH), jnp.float32),
        pltpu.VMEM((CHUNK, DEG_W), jnp.float32),
        pltpu.VMEM((CHUNK, DEG_W), jnp.float32),
        pltpu.VMEM((NBUF_ROWS, CHUNK), jnp.int32),
        pltpu.VMEM((NBUF_ROWS, CHUNK), jnp.int32),
        pltpu.VMEM((CHUNK, DH), jnp.float32),
        pltpu.VMEM((CHUNK, DH), jnp.float32),
        pltpu.SemaphoreType.DMA,
        pltpu.SemaphoreType.DMA,
        pltpu.SemaphoreType.DMA,
        pltpu.SemaphoreType.DMA,
    ],
)


def _tc_body(s_ref, d_ref, w_ref, b_ref, hu_ref, hi_ref):
    def term(e):
        d = d_ref[2 * e]
        inv = 1.0 / jnp.maximum(d, 1.0)
        m_lo = s_ref[2 * e, 0] * inv
        m_hi = s_ref[2 * e + 1, 0] * inv
        out = jnp.dot(m_lo, w_ref[e, :DH, :],
                      preferred_element_type=jnp.float32)
        out += jnp.dot(m_hi, w_ref[e, DH:, :],
                       preferred_element_type=jnp.float32)
        return out + (d > 0.0).astype(jnp.float32) * b_ref[e][None, :]

    hi_ref[...] = term(0)
    hu_ref[...] = term(1) + term(2)


_phase2 = pl.pallas_call(
    _tc_body,
    grid=(ROWS_PAD // 128,),
    in_specs=[
        pl.BlockSpec((NSLOT, 1, 128, DH), lambda b: (0, b, 0, 0)),
        pl.BlockSpec((NSLOT, 128, 1), lambda b: (0, b, 0)),
        pl.BlockSpec((3, D, D), lambda b: (0, 0, 0)),
        pl.BlockSpec((3, D), lambda b: (0, 0)),
    ],
    out_specs=[
        pl.BlockSpec((128, D), lambda b: (b, 0)),
        pl.BlockSpec((128, D), lambda b: (b, 0)),
    ],
    out_shape=[
        jax.ShapeDtypeStruct((ROWS_PAD, D), jnp.float32),
        jax.ShapeDtypeStruct((ROWS_PAD, D), jnp.float32),
    ],
)


def _split_cols(feat):
    return feat.reshape(-1, NC, DH).transpose(1, 0, 2).reshape(-1, DH)


def kernel(feat_user, feat_item, clicks_src, clicks_dst, clicked_by_src,
           clicked_by_dst, follows_src, follows_dst, W_clicks, b_clicks,
           W_clicked_by, b_clicked_by, W_follows, b_follows):
    i32 = lambda x: x.astype(jnp.int32).reshape(N_CHUNKS, CHUNK)
    zh = jnp.zeros((CHUNK, DH), jnp.float32)
    z8 = jnp.zeros((CHUNK, DEG_W), jnp.float32)
    o4 = jnp.ones((CHUNK, DEG_W), jnp.float32)
    sums, degs = _phase1(
        _split_cols(feat_user), _split_cols(feat_item),
        i32(clicks_src), i32(clicks_dst),
        i32(clicked_by_src), i32(clicked_by_dst),
        i32(follows_src), i32(follows_dst), zh, z8, o4)
    sums4 = sums.reshape(NSLOT, ROWS_PAD // 128, 128, DH)
    degs3 = degs[:, :1].reshape(NSLOT, ROWS_PAD, 1)
    wstk = jnp.stack([W_clicks, W_clicked_by, W_follows])
    bstk = jnp.stack([b_clicks, b_clicked_by, b_follows])
    h_user, h_item = _phase2(sums4, degs3, wstk, bstk)
    return (h_user[:N_NODES], h_item[:N_NODES])
